# baseline (speedup 1.0000x reference)
import functools

import jax
import jax.numpy as jnp
from jax import lax
from jax.experimental import pallas as pl
from jax.experimental.pallas import tpu as pltpu
from jax.experimental.pallas import tpu_sc as plsc

_B = 16384
_NBINS = 30
_SC_BLOCKS = 31
_CHUNK = 256
_NW = 32


def _sc_body(s_rows, c, x_hbm, lab_hbm, out_hbm,
             buf0, buf1, lbuf0, lbuf1, staging, hist,
             sem0, sem1, lsem0, lsem1):
    cid = lax.axis_index("c")
    sid = lax.axis_index("s")
    wid = sid * 2 + cid
    w_rows = s_rows // _NW
    n_chunks = w_rows // _CHUNK
    row0 = wid * w_rows

    iota = lax.iota(jnp.int32, 16)
    zeros16 = jnp.zeros((16,), jnp.float32)
    for r in range(96):
        hist[pl.ds(16 * r, 16)] = zeros16

    bufs = (buf0, buf1)
    lbufs = (lbuf0, lbuf1)
    sems = (sem0, sem1)
    lsems = (lsem0, lsem1)

    def start_copy(t, par):
        base = row0 + t * _CHUNK
        pltpu.async_copy(x_hbm.at[pl.ds(base, _CHUNK), :], bufs[par],
                         sems[par])
        pltpu.async_copy(lab_hbm.at[pl.ds(base, _CHUNK)], lbufs[par],
                         lsems[par])

    def wait_copy(par):
        pltpu.make_async_copy(x_hbm.at[pl.ds(0, _CHUNK), :], bufs[par],
                              sems[par]).wait()
        pltpu.make_async_copy(lab_hbm.at[pl.ds(0, _CHUNK)], lbufs[par],
                              lsems[par]).wait()

    sbases = [16 * j for j in range(c // 16)]
    if c % 16:
        sbases.append(c - 16)
    iota16x = iota * 16

    def process(par):
        buf = bufs[par]
        lbuf = lbufs[par]

        def group16(g, carry):
            for r in range(16):
                row = 16 * g + r
                for b in sbases:
                    vj = buf[row, pl.ds(b, 16)]
                    plsc.store_scatter(staging, [iota16x + (16 * b + r)], vj)
            m = staging[pl.ds(0, 16)]
            am = jnp.zeros((16,), jnp.int32)
            for cls in range(1, c):
                w = staging[pl.ds(16 * cls, 16)]
                upd = w > m
                am = jnp.where(upd, cls, am)
                m = jnp.maximum(m, w)
            lab16 = lbuf[pl.ds(16 * g, 16)]
            acc16 = (am == lab16).astype(jnp.float32)
            y30 = m * 30.0
            tr = y30.astype(jnp.int32)
            bin16 = tr + jnp.where(tr.astype(jnp.float32) < y30, 1, 0) - 1
            bin16 = jnp.clip(bin16, 0, _NBINS - 1)
            base16 = iota * 96 + bin16
            plsc.addupdate_scatter(hist, [base16],
                                   jnp.full((16,), 1.0, jnp.float32))
            plsc.addupdate_scatter(hist, [base16 + 32], m)
            plsc.addupdate_scatter(hist, [base16 + 64], acc16)
            return carry

        lax.fori_loop(0, _CHUNK // 16, group16, 0)

    start_copy(0, 0)
    if n_chunks > 1:
        start_copy(1, 1)

    def chunk_pair(k, carry):
        t = k * 2
        wait_copy(0)

        @pl.when(t + 2 < n_chunks)
        def _s0():
            start_copy(t + 2, 0)

        process(0)
        wait_copy(1)

        @pl.when(t + 3 < n_chunks)
        def _s1():
            start_copy(t + 3, 1)

        process(1)
        return carry

    lax.fori_loop(0, n_chunks // 2, chunk_pair, 0)

    if n_chunks % 2:
        wait_copy((n_chunks - 1) % 2)
        process((n_chunks - 1) % 2)

    pltpu.sync_copy(hist, out_hbm.at[wid])


def _sc_partials(softmaxes, labels, s_rows, c):
    mesh = plsc.VectorSubcoreMesh(core_axis_name="c", subcore_axis_name="s")
    run = pl.kernel(
        functools.partial(_sc_body, s_rows, c),
        mesh=mesh,
        compiler_params=pltpu.CompilerParams(needs_layout_passes=False),
        out_type=jax.ShapeDtypeStruct((_NW, 1536), jnp.float32),
        scratch_types=[
            pltpu.VMEM((_CHUNK, c), jnp.float32),
            pltpu.VMEM((_CHUNK, c), jnp.float32),
            pltpu.VMEM((_CHUNK,), jnp.int32),
            pltpu.VMEM((_CHUNK,), jnp.int32),
            pltpu.VMEM((16 * c,), jnp.float32),
            pltpu.VMEM((1536,), jnp.float32),
            pltpu.SemaphoreType.DMA,
            pltpu.SemaphoreType.DMA,
            pltpu.SemaphoreType.DMA,
            pltpu.SemaphoreType.DMA,
        ],
    )
    return run(softmaxes, labels)


def _tc_body(nb, n, c, off, x_ref, lab_ref, out_ref, acc_ref):
    i = pl.program_id(0)

    @pl.when(i == 0)
    def _init():
        acc_ref[...] = jnp.zeros((3, 32), jnp.float32)

    x = x_ref[...]
    xt = x.T

    conf = jnp.max(xt, axis=0, keepdims=True)
    sub_iota = lax.broadcasted_iota(jnp.int32, (c, _B), 0)
    pred = jnp.min(jnp.where(xt == conf, sub_iota, c), axis=0,
                   keepdims=True)
    lab = lab_ref[0]
    accv = (pred == lab).astype(jnp.float32)
    binv = jnp.clip(jnp.ceil(conf * _NBINS).astype(jnp.int32) - 1,
                    0, _NBINS - 1)
    rows = lax.broadcasted_iota(jnp.int32, (1, _B), 1) + (i + off) * _B
    valid = rows < n

    bin_iota = lax.broadcasted_iota(jnp.int32, (32, _B), 0)
    m = ((binv == bin_iota) & valid).astype(jnp.float32)
    y = jnp.concatenate(
        [valid.astype(jnp.float32),
         jnp.where(valid, conf, 0.0),
         jnp.where(valid, accv, 0.0)], axis=0)
    s = lax.dot_general(y, m, (((1,), (1,)), ((), ())),
                        preferred_element_type=jnp.float32)
    acc_ref[...] += s

    @pl.when(i == nb - 1)
    def _fin():
        out_ref[...] = acc_ref[...]


def _tc_partials(softmaxes, lab_p, nb_tc, n, c, off):
    return pl.pallas_call(
        functools.partial(_tc_body, nb_tc, n, c, off),
        grid=(nb_tc,),
        in_specs=[
            pl.BlockSpec((_B, c), lambda i: (i + off, 0)),
            pl.BlockSpec((1, 1, _B), lambda i: (i + off, 0, 0)),
        ],
        out_specs=pl.BlockSpec((3, 32), lambda i: (0, 0)),
        out_shape=jax.ShapeDtypeStruct((3, 32), jnp.float32),
        scratch_shapes=[pltpu.VMEM((3, 32), jnp.float32)],
    )(softmaxes, lab_p)


def _fin_body(n, tc_ref, sc_ref, out_ref):
    sc = sc_ref[...]
    s = jnp.sum(sc, axis=0)
    tcm = tc_ref[...]
    cnt = s[0:32] + tcm[0, :]
    cs = s[32:64] + tcm[1, :]
    asum = s[64:96] + tcm[2, :]
    safe = jnp.maximum(cnt, 1.0)
    gap = jnp.abs(cs / safe - asum / safe)
    gap = jnp.where(cnt > 0.0, gap, 0.0)
    ece = jnp.sum(gap * cnt) / n
    out_ref[...] = jnp.broadcast_to(ece, (1, 1))


def kernel(softmaxes, labels):
    n, c = softmaxes.shape
    nb = pl.cdiv(n, _B)
    npad = nb * _B
    lab_p = jnp.pad(labels, (0, npad - n)).reshape(nb, 1, _B)

    s_rows = _SC_BLOCKS * _B
    sc_part = _sc_partials(softmaxes, labels, s_rows, c)
    tc_part = _tc_partials(softmaxes, lab_p, nb - _SC_BLOCKS, n, c,
                           _SC_BLOCKS)

    out = pl.pallas_call(
        functools.partial(_fin_body, n),
        grid=(1,),
        in_specs=[
            pl.BlockSpec((3, 32), lambda i: (0, 0)),
            pl.BlockSpec((_NW * 16, 96), lambda i: (0, 0)),
        ],
        out_specs=pl.BlockSpec((1, 1), lambda i: (0, 0)),
        out_shape=jax.ShapeDtypeStruct((1, 1), jnp.float32),
    )(tc_part, sc_part.reshape(_NW * 16, 96))
    return out.reshape(1)
